# baseline (speedup 1.0000x reference)
import functools

import jax
import jax.numpy as jnp
from jax import lax
from jax.experimental import pallas as pl
from jax.experimental.pallas import tpu as pltpu
from jax.experimental.pallas import tpu_sc as plsc

B = 16384
D = 32


@functools.lru_cache(maxsize=None)
def _build():
    info = plsc.get_sparse_core_info()
    NC, NS, L = info.num_cores, info.num_subcores, info.num_lanes
    NW = NC * NS
    bw = B // NW
    ng = bw // L

    mesh = plsc.VectorSubcoreMesh(core_axis_name="c", subcore_axis_name="s")
    out_sds = jax.ShapeDtypeStruct((B * D,), jnp.float32)

    @functools.partial(
        pl.kernel,
        mesh=mesh,
        out_type=(out_sds, out_sds, out_sds),
        scratch_types=[
            pltpu.VMEM((bw,), jnp.int32),
            pltpu.VMEM((bw,), jnp.int32),
            pltpu.VMEM((bw,), jnp.int32),
            pltpu.VMEM((bw,), jnp.int32),
            pltpu.VMEM((bw, 4 * D), jnp.float32),
            pltpu.VMEM((bw * D,), jnp.float32),
            pltpu.SemaphoreType.DMA,
            pltpu.SemaphoreType.DMA,
            pltpu.SemaphoreType.DMA,
        ],
        compiler_params=pltpu.CompilerParams(needs_layout_passes=False),
    )
    def body(uid, iid, cid, wu, wi, wc, ou, oi, oc,
             idx_u, idx_i, idx_c, blk, buf, rows,
             sem_idx, sem_g, sem_o):
        wid = lax.axis_index("s") * NC + lax.axis_index("c")
        base = wid * bw
        sl = pl.ds(base, bw)
        osl = pl.ds(base * D, bw * D)

        for cp in [
            pltpu.async_copy(uid.at[sl], idx_u, sem_idx),
            pltpu.async_copy(iid.at[sl], idx_i, sem_idx),
            pltpu.async_copy(cid.at[sl], idx_c, sem_idx),
        ]:
            cp.wait()

        def one_table(idx_v, w_hbm, out_hbm):
            def mk_blk(g, _):
                v = idx_v[pl.ds(g * L, L)]
                blk[pl.ds(g * L, L)] = lax.shift_right_logical(v, 2)
                return _
            lax.fori_loop(0, ng, mk_blk, 0, unroll=4)

            pltpu.async_copy(w_hbm.at[blk], buf, sem_g).wait()

            def extract(g, _):
                pos = g * L + lax.iota(jnp.int32, L)
                v = idx_v[pl.ds(g * L, L)]
                colbase = lax.shift_left(jnp.bitwise_and(v, 3), 5)
                flatbase = pos * D
                for d in range(D):
                    val = plsc.load_gather(buf, [pos, colbase + d])
                    plsc.store_scatter(rows, [flatbase + d], val)
                return _
            lax.fori_loop(0, ng, extract, 0)

            pltpu.async_copy(rows, out_hbm.at[osl], sem_o).wait()

        one_table(idx_u, wu, ou)
        one_table(idx_i, wi, oi)
        one_table(idx_c, wc, oc)

    return body


def kernel(user_id, item_id, cate_id, W_user, W_item, W_cate):
    f = _build()
    zu, zi, zc = f(
        user_id.astype(jnp.int32),
        item_id.astype(jnp.int32),
        cate_id.astype(jnp.int32),
        W_user.reshape(-1, 4 * D),
        W_item.reshape(-1, 4 * D),
        W_cate.reshape(-1, 4 * D),
    )
    return (zu.reshape(B, D), zi.reshape(B, D), zc.reshape(B, D))


# speedup vs baseline: 3.8420x; 3.8420x over previous
import functools

import jax
import jax.numpy as jnp
from jax import lax
from jax.experimental import pallas as pl
from jax.experimental.pallas import tpu as pltpu
from jax.experimental.pallas import tpu_sc as plsc

B = 16384
D = 32
V_USER = 1000000
V_ITEM = 1000000
V_CATE = 100000

LANES = 128
CHUNK_V = 512
SCAN_CH = 2048
HIT_CAP = 800
STAGE_CAP = 688
CHL_CAP = 208


def _cdiv(a, b):
    return (a + b - 1) // b


@functools.lru_cache(maxsize=None)
def _build():
    info = plsc.get_sparse_core_info()
    NC, NS, L = info.num_cores, info.num_subcores, info.num_lanes
    NW = NC * NS

    mesh = plsc.VectorSubcoreMesh(core_axis_name="c", subcore_axis_name="s")
    out_sds = jax.ShapeDtypeStruct((B, LANES), jnp.float32)

    tables = []
    for V in (V_USER, V_ITEM, V_CATE):
        ncols = _cdiv(V, LANES)
        q, r = divmod(ncols, NW)
        max_cols = q + 1 if r else q
        nch = _cdiv(max_cols, CHUNK_V // LANES)
        nch += nch % 2
        tables.append((V, ncols, q, r, nch))

    @functools.partial(
        pl.kernel,
        mesh=mesh,
        out_type=(out_sds, out_sds, out_sds),
        scratch_types=[
            pltpu.VMEM((SCAN_CH,), jnp.int32),
            pltpu.VMEM((HIT_CAP,), jnp.int32),
            pltpu.VMEM((HIT_CAP,), jnp.int32),
            pltpu.VMEM((CHL_CAP,), jnp.int32),
            pltpu.VMEM((CHL_CAP,), jnp.int32),
            pltpu.VMEM((D, CHUNK_V), jnp.float32),
            pltpu.VMEM((D, CHUNK_V), jnp.float32),
            pltpu.VMEM((STAGE_CAP, LANES), jnp.float32),
            pltpu.VMEM((STAGE_CAP,), jnp.int32),
            pltpu.SemaphoreType.DMA,
            pltpu.SemaphoreType.DMA,
            pltpu.SemaphoreType.DMA,
        ],
        compiler_params=pltpu.CompilerParams(needs_layout_passes=False),
    )
    def body(uid, iid, cid, wu, wi, wc, ou, oi, oc,
             scanbuf, hv, hb, cv, cb, chunk0, chunk1, stage, blist,
             sem_scan, sem_chunk, sem_scat):
        w = lax.axis_index("s") * NC + lax.axis_index("c")
        iota = lax.iota(jnp.int32, L)
        chunks = (chunk0, chunk1)

        def one_table(idx_hbm, wt_hbm, out_hbm, cfg, first):
            V, ncols, q, r, nch = cfg
            my_cols = jnp.minimum(w, r) + w * q
            lo = my_cols * LANES
            hi = lo + (q + jnp.where(w < r, 1, 0)) * LANES

            def scan_chunk(sc, ptr):
                pltpu.async_copy(
                    idx_hbm.at[pl.ds(sc * SCAN_CH, SCAN_CH)], scanbuf,
                    sem_scan).wait()

                def scan_group(g, ptr):
                    v16 = scanbuf[pl.ds(g * L, L)]
                    m = jnp.logical_and(v16 >= lo, v16 < hi)
                    mi = m.astype(jnp.int32)
                    cum = plsc.cumsum(mi)
                    rows = ptr + cum - mi
                    rows = jnp.where(m, rows, 0)
                    b16 = sc * SCAN_CH + g * L + iota
                    plsc.store_scatter(hv, [rows], v16, mask=m)
                    plsc.store_scatter(hb, [rows], b16, mask=m)
                    return ptr + cum[L - 1]

                return lax.fori_loop(0, SCAN_CH // L, scan_group, ptr)

            nhit = lax.fori_loop(0, B // SCAN_CH, scan_chunk, 0)
            ngrp = lax.div(nhit + L - 1, L)

            neg = jnp.full((L,), -1, jnp.int32)

            def prefill(s, _):
                blist[pl.ds(s * L, L)] = neg
                return _

            lax.fori_loop(0, STAGE_CAP // L, prefill, 0)

            max_start = (ncols * LANES) - CHUNK_V

            def chunk_start(k):
                return pl.multiple_of(
                    jnp.minimum(lo + k * CHUNK_V, max_start), LANES)

            def fire(k, buf):
                pltpu.async_copy(
                    wt_hbm.at[:, pl.ds(chunk_start(k), CHUNK_V)], buf,
                    sem_chunk)

            fire(0, chunk0)
            fire(1, chunk1)

            def do_chunk(k, buf, cnt):
                vb = chunk_start(k)
                pltpu.make_async_copy(
                    wt_hbm.at[:, pl.ds(vb, CHUNK_V)], buf, sem_chunk).wait()

                def match(j, cptr):
                    v16 = hv[pl.ds(j * L, L)]
                    b16 = hb[pl.ds(j * L, L)]
                    valid = (j * L + iota) < nhit
                    m = jnp.logical_and(
                        valid,
                        jnp.logical_and(v16 >= vb, v16 < vb + CHUNK_V))
                    mi = m.astype(jnp.int32)
                    cum = plsc.cumsum(mi)
                    rows = jnp.where(m, cptr + cum - mi, 0)
                    plsc.store_scatter(cv, [rows], v16 - vb, mask=m)
                    plsc.store_scatter(cb, [rows], b16, mask=m)
                    return cptr + cum[L - 1]

                cptr = lax.fori_loop(0, ngrp, match, 0)

                def extract(j, cnt):
                    cols = cv[pl.ds(j * L, L)]
                    pos = cb[pl.ds(j * L, L)]
                    m = (j * L + iota) < cptr
                    cols = jnp.where(m, cols, 0)
                    rows = jnp.where(m, cnt + iota, 0)
                    plsc.store_scatter(blist, [rows], pos, mask=m)
                    for d in range(D):
                        dvec = jnp.full((L,), d, jnp.int32)
                        val = plsc.load_gather(buf, [dvec, cols], mask=m)
                        plsc.store_scatter(stage, [rows, dvec], val, mask=m)
                    take = jnp.minimum(cptr - j * L, L)
                    return cnt + take

                return lax.fori_loop(0, lax.div(cptr + L - 1, L), extract,
                                     cnt)

            def do_pair(k2, cnt):
                k = k2 * 2
                cnt = do_chunk(k, chunk0, cnt)
                fire(k + 2, chunk0)
                cnt = do_chunk(k + 1, chunk1, cnt)
                fire(k + 3, chunk1)
                return cnt

            lax.fori_loop(0, nch // 2, do_pair, 0)

            pltpu.make_async_copy(
                wt_hbm.at[:, pl.ds(chunk_start(0), CHUNK_V)], chunk0,
                sem_chunk).wait()
            pltpu.make_async_copy(
                wt_hbm.at[:, pl.ds(chunk_start(0), CHUNK_V)], chunk1,
                sem_chunk).wait()

            cp = pltpu.async_copy(
                stage,
                out_hbm.at[plsc.Indices(blist, ignored_value=-1)],
                sem_scat)
            return cp

        cfg_u, cfg_i, cfg_c = tables
        cp1 = one_table(uid, wu, ou, cfg_u, True)
        cp1.wait()
        cp2 = one_table(iid, wi, oi, cfg_i, False)
        cp2.wait()
        cp3 = one_table(cid, wc, oc, cfg_c, False)
        cp3.wait()

    return body


def kernel(user_id, item_id, cate_id, W_user, W_item, W_cate):
    f = _build()
    zu, zi, zc = f(
        user_id.astype(jnp.int32),
        item_id.astype(jnp.int32),
        cate_id.astype(jnp.int32),
        W_user.T,
        W_item.T,
        W_cate.T,
    )
    return (zu[:, :D], zi[:, :D], zc[:, :D])
